# Initial kernel scaffold; baseline (speedup 1.0000x reference)
import jax
import jax.numpy as jnp
from jax.experimental import pallas as pl


def kernel(x, Wr, W1, b1, W2, b2):
    raise NotImplementedError("write your pallas kernel here")


# speedup vs baseline: 2.3211x; 2.3211x over previous
import functools
import math

import jax
import jax.numpy as jnp
from jax.experimental import pallas as pl


def _router_body(x_ref, wr_ref, maskT_ref):
    xb = x_ref[...]
    wr = wr_ref[...]
    logits = jax.lax.dot_general(
        xb, wr, (((1,), (1,)), ((), ())), preferred_element_type=jnp.float32
    )
    bt, e = logits.shape
    iota_t = jax.lax.broadcasted_iota(jnp.int32, (e, bt), 0)
    i1 = jnp.argmax(logits, axis=1)
    m1 = jnp.max(logits, axis=1)
    masked = jnp.where(iota_t.T == i1[:, None], -jnp.inf, logits)
    i2 = jnp.argmax(masked, axis=1)
    m2 = jnp.max(masked, axis=1)
    w0 = 1.0 / (1.0 + jnp.exp(m2 - m1))
    w1 = 1.0 - w0
    maskT = jnp.where(iota_t == i1[None, :], w0[None, :], 0.0) + jnp.where(
        iota_t == i2[None, :], w1[None, :], 0.0
    )
    maskT_ref[...] = maskT[:, None, :]


def _ffn_body(x_ref, w1_ref, b1_ref, w2_ref, b2_ref, maskT_ref, out_ref):
    e = pl.program_id(1)
    dc = pl.program_id(2)
    xb = x_ref[...]
    w1 = w1_ref[0]
    h = jax.lax.dot_general(
        xb, w1, (((1,), (1,)), ((), ())), preferred_element_type=jnp.float32
    ) + b1_ref[0]
    h = 0.5 * h * (1.0 + jax.lax.erf(h * (1.0 / math.sqrt(2.0))))
    w2 = w2_ref[0]
    o = jax.lax.dot_general(
        h, w2, (((1,), (1,)), ((), ())), preferred_element_type=jnp.float32
    )
    mcol = maskT_ref[0, 0, :][:, None]
    part = o * mcol

    @pl.when((e == 0) & (dc == 0))
    def _init():
        out_ref[...] = part + mcol * b2_ref[0]

    @pl.when(dc == 0)
    def _bias():
        @pl.when(e > 0)
        def _():
            out_ref[...] = out_ref[...] + part + mcol * b2_ref[0]

    @pl.when(dc > 0)
    def _acc():
        out_ref[...] = out_ref[...] + part


def _moe_dense(x_flat, Wr, W1, b1, W2, b2, *, interpret=False):
    n, c = x_flat.shape
    e_num, d = W1.shape[0], W1.shape[1]
    bt = min(512, n)
    nb = n // bt

    maskT = pl.pallas_call(
        _router_body,
        grid=(nb,),
        in_specs=[
            pl.BlockSpec((bt, c), lambda tb: (tb, 0)),
            pl.BlockSpec((e_num, c), lambda tb: (0, 0)),
        ],
        out_specs=pl.BlockSpec((e_num, 1, bt), lambda tb: (0, 0, tb)),
        out_shape=jax.ShapeDtypeStruct((e_num, 1, n), jnp.float32),
        interpret=interpret,
    )(x_flat, Wr)

    b1r = b1[:, None, :]
    b2r = b2[:, None, :]
    dcb = min(1024, d)
    ndc = d // dcb
    out = pl.pallas_call(
        _ffn_body,
        grid=(nb, e_num, ndc),
        in_specs=[
            pl.BlockSpec((bt, c), lambda tb, e, dc: (tb, 0)),
            pl.BlockSpec((1, dcb, c), lambda tb, e, dc: (e, dc, 0)),
            pl.BlockSpec((1, 1, dcb), lambda tb, e, dc: (e, 0, dc)),
            pl.BlockSpec((1, c, dcb), lambda tb, e, dc: (e, 0, dc)),
            pl.BlockSpec((1, 1, c), lambda tb, e, dc: (e, 0, 0)),
            pl.BlockSpec((1, 1, bt), lambda tb, e, dc: (e, 0, tb)),
        ],
        out_specs=pl.BlockSpec((bt, c), lambda tb, e, dc: (tb, 0)),
        out_shape=jax.ShapeDtypeStruct((n, c), jnp.float32),
        interpret=interpret,
    )(x_flat, W1, b1r, W2, b2r, maskT)
    return out


def kernel(x, Wr, W1, b1, W2, b2):
    bx, tx, cx = x.shape
    x_flat = x.reshape(bx * tx, cx)
    out = _moe_dense(x_flat, Wr, W1, b1, W2, b2)
    aux_loss = jnp.zeros((), dtype=x.dtype)
    return (out.reshape(bx, tx, cx), aux_loss)
